# Initial kernel scaffold; baseline (speedup 1.0000x reference)
import jax
import jax.numpy as jnp
from jax.experimental import pallas as pl


def kernel(users, items, user_emb, item_emb, adj_rows, adj_cols, adj_vals):
    raise NotImplementedError("write your pallas kernel here")


# speedup vs baseline: 3.4509x; 3.4509x over previous
import functools

import jax
import jax.numpy as jnp
from jax import lax
from jax.experimental import pallas as pl
from jax.experimental.pallas import tpu as pltpu
from jax.experimental.pallas import tpu_sc as plsc

_NU = 50000
_NN = 100000
_NP = 100352
_D = 32
_H = 16
_L = 16
_NL = 3
_E = 1600000
_B = 4096
_EB = 128
_NB = 12800
_EPAD = _NB * _EB
_NC = 2
_NS = 16
_BPT = _NB // _NS
_G8 = 4
_NG = _BPT // _G8
_RPT = _NP // _NS
_ZR = 196
_PPT = _B // _NS
_FR = _B // _EB

_mesh = plsc.VectorSubcoreMesh(
    core_axis_name="c", subcore_axis_name="s", num_cores=_NC, num_subcores=_NS
)


@functools.partial(
    pl.kernel,
    out_type=(
        jax.ShapeDtypeStruct((_NC * _B,), jnp.float32),
        jax.ShapeDtypeStruct((_NC * _NP, _H), jnp.float32)
    ),
    mesh=_mesh,
    compiler_params=pltpu.CompilerParams(
        needs_layout_passes=False, use_tc_tiling_on_sc=False
    ),
    scratch_types=[
        pltpu.VMEM_SHARED((_NP, _H), jnp.float32),
        pltpu.VMEM((_EB,), jnp.int32),
        pltpu.VMEM((_EB,), jnp.int32),
        pltpu.VMEM((_EB,), jnp.float32),
        pltpu.VMEM((_EB, _H), jnp.float32),
        pltpu.VMEM((_EB,), jnp.int32),
        pltpu.VMEM((_EB, _H), jnp.float32),
        pltpu.VMEM((2 * _EB, _H), jnp.float32),
        pltpu.VMEM((2 * _EB, _H), jnp.float32),
        pltpu.VMEM((_PPT,), jnp.float32),
        pltpu.VMEM((_ZR, _H), jnp.float32),
        pltpu.SemaphoreType.DMA,
    ],
)
def _lightgcn_sc(emb0, cols, rows, vals, fnodes, out, ebuf,
                 accum, cols_v, rows_v, vals_v, gbuf, fidx_v,
                 tmp_v, usum, isum, outv, zbuf, gsem):
    c = lax.axis_index("c")
    s = lax.axis_index("s")
    zero16 = jnp.zeros((_L,), jnp.float32)
    iota = lax.iota(jnp.int32, _L)

    def _zero_slab(r, _):
        zbuf[r, :] = zero16
        return 0
    lax.fori_loop(0, _ZR, _zero_slab, 0, unroll=8)

    def _zero_sums(r, _):
        usum[r, :] = zero16
        isum[r, :] = zero16
        return 0
    lax.fori_loop(0, 2 * _EB, _zero_sums, 0, unroll=8)

    row0 = s * _RPT
    for kk in range(_RPT // _ZR):
        pltpu.sync_copy(zbuf, accum.at[pl.ds(row0 + kk * _ZR, _ZR)])

    def _acc_final(src):
        for half, dst in ((0, usum), (1, isum)):
            for r in range(2):
                pltpu.sync_copy(fnodes.at[pl.ds((c * 2 * _FR + half * _FR + 2 * s + r) * _EB, _EB)], fidx_v)
                pltpu.async_copy(src.at[fidx_v], tmp_v, gsem).wait()

                def _add(e, _, dst=dst, r=r):
                    dst[r * _EB + e, :] = dst[r * _EB + e, :] + tmp_v[e, :]
                    return 0
                lax.fori_loop(0, _EB, _add, 0, unroll=8)

    _acc_final(emb0)
    plsc.subcore_barrier()

    blk0 = s * _BPT
    for layer in range(_NL):
        src = emb0 if layer == 0 else ebuf

        def _block(b, _, src=src):
            blk = blk0 + b
            pltpu.sync_copy(cols.at[pl.ds(c * _EPAD + blk * _EB, _EB)], cols_v)
            pltpu.sync_copy(rows.at[pl.ds(blk * _EB, _EB)], rows_v)
            pltpu.sync_copy(vals.at[pl.ds(blk * _EB, _EB)], vals_v)
            pltpu.async_copy(src.at[cols_v], gbuf, gsem).wait()

            def _scale(t, _):
                base16 = t * _L
                for e in range(_L):
                    idx = jnp.full((_L,), base16 + e, jnp.int32)
                    bc = plsc.load_gather(vals_v, [idx])
                    gbuf[base16 + e, :] = gbuf[base16 + e, :] * bc
                return 0
            lax.fori_loop(0, _EB // _L, _scale, 0)
            pltpu.sync_copy(gbuf, accum.at[rows_v], add=True)
            return 0
        lax.fori_loop(0, _BPT, _block, 0)
        plsc.subcore_barrier()

        for kk in range(_RPT // _ZR):
            off = row0 + kk * _ZR
            pltpu.sync_copy(accum.at[pl.ds(off, _ZR)],
                            ebuf.at[pl.ds(c * _NP + off, _ZR)])
            pltpu.sync_copy(zbuf, accum.at[pl.ds(off, _ZR)])
        plsc.subcore_barrier()
        _acc_final(ebuf)

    def _dot(g, _):
        res = zero16
        for e in range(_L):
            p = g * _L + e
            prod = usum[p, :] * isum[p, :]
            sv = jnp.sum(prod)
            res = jnp.where(iota == e, sv, res)
        outv[pl.ds(g * _L, _L)] = res * (1.0 / 16.0)
        return 0
    lax.fori_loop(0, _PPT // _L, _dot, 0)
    pltpu.sync_copy(outv, out.at[pl.ds(c * _B + s * _PPT, _PPT)])


def kernel(users, items, user_emb, item_emb, adj_rows, adj_cols, adj_vals):
    emb0 = jnp.concatenate([user_emb, item_emb,
                            jnp.zeros((_NP - _NN, _D), jnp.float32)], axis=0)
    emb0h = jnp.concatenate([emb0[:, :_H], emb0[:, _H:]], axis=0)
    pad = _EPAD - _E
    cols = jnp.concatenate([adj_cols.astype(jnp.int32),
                            jnp.zeros((pad,), jnp.int32)])
    colsc = jnp.concatenate([cols, cols + _NP])
    rows = jnp.concatenate([adj_rows.astype(jnp.int32),
                            jnp.zeros((pad,), jnp.int32)])
    vals = jnp.concatenate([adj_vals,
                            jnp.zeros((pad,), jnp.float32)])
    fn = jnp.concatenate([users.astype(jnp.int32),
                          items.astype(jnp.int32) + _NU])
    fnc = jnp.concatenate([fn, fn + _NP])
    out, _ = _lightgcn_sc(emb0h, colsc, rows, vals, fnc)
    o = out.reshape(_NC, _B)
    return o[0] + o[1]


# speedup vs baseline: 6.5642x; 1.9022x over previous
import functools

import jax
import jax.numpy as jnp
from jax import lax
from jax.experimental import pallas as pl
from jax.experimental.pallas import tpu as pltpu
from jax.experimental.pallas import tpu_sc as plsc

_NU = 50000
_NN = 100000
_NP = 100352
_D = 32
_H = 16
_L = 16
_NL = 3
_E = 1600000
_B = 4096
_EB = 128
_NB = 12800
_EPAD = _NB * _EB
_NC = 2
_NS = 16
_BPT = _NB // _NS
_NPAIR = _BPT // 2
_RPT = _NP // _NS
_ZR = 196
_PPT = _B // _NS
_FR = _B // _EB

_mesh = plsc.VectorSubcoreMesh(
    core_axis_name="c", subcore_axis_name="s", num_cores=_NC, num_subcores=_NS
)


@functools.partial(
    pl.kernel,
    out_type=(
        jax.ShapeDtypeStruct((_NC * _B,), jnp.float32),
        jax.ShapeDtypeStruct((_NC * _NP, _H), jnp.float32)
    ),
    mesh=_mesh,
    compiler_params=pltpu.CompilerParams(
        needs_layout_passes=False, use_tc_tiling_on_sc=False
    ),
    scratch_types=[
        pltpu.VMEM_SHARED((_NP, _H), jnp.float32),
        pltpu.VMEM((_EB,), jnp.int32),
        pltpu.VMEM((_EB,), jnp.int32),
        pltpu.VMEM((_EB,), jnp.float32),
        pltpu.VMEM((_EB,), jnp.int32),
        pltpu.VMEM((_EB,), jnp.int32),
        pltpu.VMEM((_EB,), jnp.float32),
        pltpu.VMEM((_EB, _H), jnp.float32),
        pltpu.VMEM((_EB, _H), jnp.float32),
        pltpu.VMEM((_EB,), jnp.int32),
        pltpu.VMEM((_EB, _H), jnp.float32),
        pltpu.VMEM((2 * _EB, _H), jnp.float32),
        pltpu.VMEM((2 * _EB, _H), jnp.float32),
        pltpu.VMEM((_PPT,), jnp.float32),
        pltpu.VMEM((_ZR, _H), jnp.float32),
        pltpu.SemaphoreType.DMA,
        pltpu.SemaphoreType.DMA,
        pltpu.SemaphoreType.DMA,
        pltpu.SemaphoreType.DMA,
    ],
)
def _lightgcn_sc(emb0, cols, rows, vals, fnodes, out, ebuf,
                 accum, cols_v0, rows_v0, vals_v0, cols_v1, rows_v1, vals_v1,
                 gbuf0, gbuf1, fidx_v, tmp_v, usum, isum, outv, zbuf,
                 isem0, isem1, gsem, ssem):
    c = lax.axis_index("c")
    s = lax.axis_index("s")
    zero16 = jnp.zeros((_L,), jnp.float32)
    iota = lax.iota(jnp.int32, _L)

    def _zero_slab(r, _):
        zbuf[r, :] = zero16
        return 0
    lax.fori_loop(0, _ZR, _zero_slab, 0, unroll=8)

    def _zero_sums(r, _):
        usum[r, :] = zero16
        isum[r, :] = zero16
        return 0
    lax.fori_loop(0, 2 * _EB, _zero_sums, 0, unroll=8)

    row0 = s * _RPT
    for kk in range(_RPT // _ZR):
        pltpu.sync_copy(zbuf, accum.at[pl.ds(row0 + kk * _ZR, _ZR)])

    def _acc_final(src):
        for half, dst in ((0, usum), (1, isum)):
            for r in range(2):
                pltpu.sync_copy(
                    fnodes.at[pl.ds(
                        (c * 2 * _FR + half * _FR + 2 * s + r) * _EB, _EB)],
                    fidx_v)
                pltpu.async_copy(src.at[fidx_v], tmp_v, gsem).wait()

                def _add(e, _, dst=dst, r=r):
                    dst[r * _EB + e, :] = dst[r * _EB + e, :] + tmp_v[e, :]
                    return 0
                lax.fori_loop(0, _EB, _add, 0, unroll=8)

    _acc_final(emb0)
    plsc.subcore_barrier()

    def _scale(gbuf, vals_v):
        def _scale16(t, _):
            base16 = t * _L
            for e in range(_L):
                idx = jnp.full((_L,), base16 + e, jnp.int32)
                bc = plsc.load_gather(vals_v, [idx])
                gbuf[base16 + e, :] = gbuf[base16 + e, :] * bc
            return 0
        lax.fori_loop(0, _EB // _L, _scale16, 0)

    blk0 = s * _BPT
    for layer in range(_NL):
        src = emb0 if layer == 0 else ebuf

        def _pair(g, _, src=src):
            e0 = (blk0 + 2 * g) * _EB
            e1 = e0 + _EB
            dc0 = pltpu.async_copy(cols.at[pl.ds(c * _EPAD + e0, _EB)],
                                   cols_v0, isem0)
            dr0 = pltpu.async_copy(rows.at[pl.ds(e0, _EB)], rows_v0, isem0)
            dv0 = pltpu.async_copy(vals.at[pl.ds(e0, _EB)], vals_v0, isem0)
            dc1 = pltpu.async_copy(cols.at[pl.ds(c * _EPAD + e1, _EB)],
                                   cols_v1, isem1)
            dr1 = pltpu.async_copy(rows.at[pl.ds(e1, _EB)], rows_v1, isem1)
            dv1 = pltpu.async_copy(vals.at[pl.ds(e1, _EB)], vals_v1, isem1)
            dc0.wait(); dr0.wait(); dv0.wait()
            dg0 = pltpu.async_copy(src.at[cols_v0], gbuf0, gsem)
            dc1.wait(); dr1.wait(); dv1.wait()
            dg1 = pltpu.async_copy(src.at[cols_v1], gbuf1, gsem)
            dg0.wait()
            _scale(gbuf0, vals_v0)
            ds0 = pltpu.async_copy(gbuf0, accum.at[rows_v0], ssem, add=True)
            dg1.wait()
            _scale(gbuf1, vals_v1)
            ds0.wait()
            pltpu.sync_copy(gbuf1, accum.at[rows_v1], add=True)
            return 0
        lax.fori_loop(0, _NPAIR, _pair, 0)
        plsc.subcore_barrier()

        for kk in range(_RPT // _ZR):
            off = row0 + kk * _ZR
            pltpu.sync_copy(accum.at[pl.ds(off, _ZR)],
                            ebuf.at[pl.ds(c * _NP + off, _ZR)])
            pltpu.sync_copy(zbuf, accum.at[pl.ds(off, _ZR)])
        plsc.subcore_barrier()
        _acc_final(ebuf)

    def _dot(g, _):
        res = zero16
        for e in range(_L):
            p = g * _L + e
            prod = usum[p, :] * isum[p, :]
            sv = jnp.sum(prod)
            res = jnp.where(iota == e, sv, res)
        outv[pl.ds(g * _L, _L)] = res * (1.0 / 16.0)
        return 0
    lax.fori_loop(0, _PPT // _L, _dot, 0)
    pltpu.sync_copy(outv, out.at[pl.ds(c * _B + s * _PPT, _PPT)])


def kernel(users, items, user_emb, item_emb, adj_rows, adj_cols, adj_vals):
    emb0 = jnp.concatenate([user_emb, item_emb,
                            jnp.zeros((_NP - _NN, _D), jnp.float32)], axis=0)
    emb0h = jnp.concatenate([emb0[:, :_H], emb0[:, _H:]], axis=0)
    pad = _EPAD - _E
    cols = jnp.concatenate([adj_cols.astype(jnp.int32),
                            jnp.zeros((pad,), jnp.int32)])
    colsc = jnp.concatenate([cols, cols + _NP])
    rows = jnp.concatenate([adj_rows.astype(jnp.int32),
                            jnp.zeros((pad,), jnp.int32)])
    vals = jnp.concatenate([adj_vals,
                            jnp.zeros((pad,), jnp.float32)])
    fn = jnp.concatenate([users.astype(jnp.int32),
                          items.astype(jnp.int32) + _NU])
    fnc = jnp.concatenate([fn, fn + _NP])
    out, _ = _lightgcn_sc(emb0h, colsc, rows, vals, fnc)
    o = out.reshape(_NC, _B)
    return o[0] + o[1]
